# baseline (device time: 16652 ns/iter reference)
import jax
import jax.numpy as jnp
from jax import lax
from jax.experimental import pallas as pl
from jax.experimental.pallas import tpu as pltpu

N_DEV = 16
NPLANE = 4
NQ = 4
BLK = 128
PRE = 2
CHUNK_A = 11

_DEV_ID_TYPE = getattr(pltpu, "DeviceIdType", None) or pl.DeviceIdType


def kernel(x):
    m, n = x.shape
    n_blk = m // BLK

    def body(
        x_ref,
        out_ref,
        total_ref,
        ptot_ref,
        intra_ref,
        inter_ref,
        ready_sems,
        send_intra,
        recv_intra,
        send_inter,
        recv_inter,
    ):
        me = lax.axis_index("i")
        zp = me // NQ
        q = me % NQ

        barrier_sem = pltpu.get_barrier_semaphore()
        for dq in range(1, NQ):
            pl.semaphore_signal(
                barrier_sem,
                inc=1,
                device_id=(zp * NQ + (q + dq) % NQ,),
                device_id_type=_DEV_ID_TYPE.MESH,
            )

        for w in range(1, NPLANE):

            @pl.when(zp == w)
            def _ready(w=w):
                for z in range(w):
                    pl.semaphore_signal(
                        ready_sems.at[w],
                        inc=1,
                        device_id=(z * NQ + q,),
                        device_id_type=_DEV_ID_TYPE.MESH,
                    )

        total_ref[0, :] = jnp.sum(x_ref[:, :], axis=0)
        intra_ref[q, :] = total_ref[0, :]

        r = lax.broadcasted_iota(jnp.int32, (BLK, BLK), 0)
        c = lax.broadcasted_iota(jnp.int32, (BLK, BLK), 1)
        tri = (r >= c).astype(jnp.bfloat16)

        def cumsum_block(g, off):
            blk = x_ref[pl.ds(g * BLK, BLK), :].astype(jnp.bfloat16)
            cs = jax.lax.dot(tri, blk, preferred_element_type=jnp.float32)
            out_ref[pl.ds(g * BLK, BLK), :] = (cs + off).astype(jnp.bfloat16)
            return off + cs[BLK - 1 : BLK, :]

        off = jnp.zeros((1, n), jnp.float32)
        for g in range(PRE):
            off = cumsum_block(g, off)

        pl.semaphore_wait(barrier_sem, NQ - 1)
        for dq in range(1, NQ):
            qp = (q + dq) % NQ
            rdma = pltpu.make_async_remote_copy(
                src_ref=total_ref.at[0],
                dst_ref=intra_ref.at[q],
                send_sem=send_intra.at[qp],
                recv_sem=recv_intra.at[q],
                device_id=(zp * NQ + qp,),
                device_id_type=_DEV_ID_TYPE.MESH,
            )
            rdma.start()

        for g in range(PRE, PRE + CHUNK_A):
            off = cumsum_block(g, off)

        for dq in range(1, NQ):
            qp = (q + dq) % NQ
            rdma = pltpu.make_async_remote_copy(
                src_ref=total_ref.at[0],
                dst_ref=intra_ref.at[qp],
                send_sem=send_intra.at[qp],
                recv_sem=recv_intra.at[qp],
                device_id=(0,),
                device_id_type=_DEV_ID_TYPE.MESH,
            )
            rdma.wait_recv()

        intra = intra_ref[:, :]
        ptot_ref[0, :] = jnp.sum(intra, axis=0)
        q_ids = lax.broadcasted_iota(jnp.int32, (NQ, n), 0)
        in_plane = jnp.sum(
            jnp.where(q_ids < q, intra, jnp.zeros_like(intra)),
            axis=0,
            keepdims=True,
        )

        for z in range(NPLANE - 1):

            @pl.when(zp == z)
            def _send_up(z=z):
                for w in range(z + 1, NPLANE):
                    pl.semaphore_wait(ready_sems.at[w], 1)
                    rdma = pltpu.make_async_remote_copy(
                        src_ref=ptot_ref.at[0],
                        dst_ref=inter_ref.at[z],
                        send_sem=send_inter.at[w],
                        recv_sem=recv_inter.at[z],
                        device_id=(w * NQ + q,),
                        device_id_type=_DEV_ID_TYPE.MESH,
                    )
                    rdma.start()

        for g in range(PRE + CHUNK_A, n_blk):
            off = cumsum_block(g, off)

        for z in range(NPLANE - 1):

            @pl.when(z < zp)
            def _recv_down(z=z):
                rdma = pltpu.make_async_remote_copy(
                    src_ref=ptot_ref.at[0],
                    dst_ref=inter_ref.at[z],
                    send_sem=send_inter.at[z],
                    recv_sem=recv_inter.at[z],
                    device_id=(0,),
                    device_id_type=_DEV_ID_TYPE.MESH,
                )
                rdma.wait_recv()

        inter = inter_ref[:, :]
        z_ids = lax.broadcasted_iota(jnp.int32, (NPLANE, n), 0)
        offset = in_plane + jnp.sum(
            jnp.where(z_ids < zp, inter, jnp.zeros_like(inter)),
            axis=0,
            keepdims=True,
        )
        offset16 = offset.astype(jnp.bfloat16)

        for g in range(n_blk):
            out_ref[pl.ds(g * BLK, BLK), :] = (
                out_ref[pl.ds(g * BLK, BLK), :] + offset16
            )

        for dq in range(1, NQ):
            qp = (q + dq) % NQ
            rdma = pltpu.make_async_remote_copy(
                src_ref=total_ref.at[0],
                dst_ref=intra_ref.at[q],
                send_sem=send_intra.at[qp],
                recv_sem=recv_intra.at[q],
                device_id=(zp * NQ + qp,),
                device_id_type=_DEV_ID_TYPE.MESH,
            )
            rdma.wait_send()
        for z in range(NPLANE - 1):

            @pl.when(zp == z)
            def _drain_up(z=z):
                for w in range(z + 1, NPLANE):
                    rdma = pltpu.make_async_remote_copy(
                        src_ref=ptot_ref.at[0],
                        dst_ref=inter_ref.at[z],
                        send_sem=send_inter.at[w],
                        recv_sem=recv_inter.at[z],
                        device_id=(w * NQ + q,),
                        device_id_type=_DEV_ID_TYPE.MESH,
                    )
                    rdma.wait_send()

    return pl.pallas_call(
        body,
        out_shape=jax.ShapeDtypeStruct((m, n), jnp.bfloat16),
        in_specs=[pl.BlockSpec(memory_space=pltpu.VMEM)],
        out_specs=pl.BlockSpec(memory_space=pltpu.VMEM),
        scratch_shapes=[
            pltpu.VMEM((1, n), jnp.float32),
            pltpu.VMEM((1, n), jnp.float32),
            pltpu.VMEM((NQ, n), jnp.float32),
            pltpu.VMEM((NPLANE, n), jnp.float32),
            pltpu.SemaphoreType.REGULAR((NPLANE,)),
            pltpu.SemaphoreType.DMA((NQ,)),
            pltpu.SemaphoreType.DMA((NQ,)),
            pltpu.SemaphoreType.DMA((NPLANE,)),
            pltpu.SemaphoreType.DMA((NPLANE,)),
        ],
        compiler_params=pltpu.CompilerParams(collective_id=0),
    )(x)


# device time: 13801 ns/iter; 1.2066x vs baseline; 1.2066x over previous
import jax
import jax.numpy as jnp
from jax import lax
from jax.experimental import pallas as pl
from jax.experimental.pallas import tpu as pltpu

N_DEV = 16
BLK = 128
N_CHUNK = 8
N_OGRP = 4

_DEV_ID_TYPE = getattr(pltpu, "DeviceIdType", None) or pl.DeviceIdType


def kernel(x):
    m, n = x.shape
    n_blk = m // BLK
    crows = m // N_CHUNK
    orows = m // N_OGRP

    def body(
        x_ref,
        out_ref,
        stage_ref,
        outv_ref,
        total_ref,
        comm_ref,
        in_sems,
        out_sems,
        send_sems,
        recv_sems,
    ):
        me = lax.axis_index("i")

        barrier_sem = pltpu.get_barrier_semaphore()
        for p in range(N_DEV):

            @pl.when(me != p)
            def _signal(p=p):
                pl.semaphore_signal(
                    barrier_sem,
                    inc=1,
                    device_id=(p,),
                    device_id_type=_DEV_ID_TYPE.MESH,
                )

        def in_copy(k):
            return pltpu.make_async_copy(
                x_ref.at[pl.ds(k * crows, crows), :],
                stage_ref.at[pl.ds(k * crows, crows), :],
                in_sems.at[k],
            )

        for k in range(N_CHUNK):
            in_copy(k).start()

        r = lax.broadcasted_iota(jnp.int32, (BLK, BLK), 0)
        c = lax.broadcasted_iota(jnp.int32, (BLK, BLK), 1)
        tri = (r >= c).astype(jnp.bfloat16)

        def cumsum_block(g, off):
            blk = stage_ref[pl.ds(g * BLK, BLK), :].astype(jnp.bfloat16)
            cs = jax.lax.dot(tri, blk, preferred_element_type=jnp.float32)
            outv_ref[pl.ds(g * BLK, BLK), :] = (cs + off).astype(jnp.bfloat16)
            return off + cs[BLK - 1 : BLK, :]

        off = jnp.zeros((1, n), jnp.float32)
        tot = jnp.zeros((1, n), jnp.float32)
        bpc = crows // BLK
        for k in range(N_CHUNK):
            in_copy(k).wait()
            tot = tot + jnp.sum(
                stage_ref[pl.ds(k * crows, crows), :], axis=0, keepdims=True
            )
            for b in range(bpc):
                off = cumsum_block(k * bpc + b, off)
        total_ref[0, :] = tot[0, :]

        pl.semaphore_wait(barrier_sem, N_DEV - 1)

        for j in range(1, N_DEV):

            @pl.when(me < j)
            def _send(j=j):
                rdma = pltpu.make_async_remote_copy(
                    src_ref=total_ref.at[0],
                    dst_ref=comm_ref.at[me],
                    send_sem=send_sems.at[j],
                    recv_sem=recv_sems.at[me],
                    device_id=(j,),
                    device_id_type=_DEV_ID_TYPE.MESH,
                )
                rdma.start()

        for k in range(N_DEV - 1):

            @pl.when(k < me)
            def _recv(k=k):
                rdma = pltpu.make_async_remote_copy(
                    src_ref=total_ref.at[0],
                    dst_ref=comm_ref.at[k],
                    send_sem=send_sems.at[k],
                    recv_sem=recv_sems.at[k],
                    device_id=(0,),
                    device_id_type=_DEV_ID_TYPE.MESH,
                )
                rdma.wait_recv()

        row_ids = lax.broadcasted_iota(jnp.int32, (N_DEV, n), 0)
        comm = comm_ref[:, :]
        offset16 = jnp.sum(
            jnp.where(row_ids < me, comm, jnp.zeros_like(comm)),
            axis=0,
            keepdims=True,
        ).astype(jnp.bfloat16)

        def out_copy(grp):
            return pltpu.make_async_copy(
                outv_ref.at[pl.ds(grp * orows, orows), :],
                out_ref.at[pl.ds(grp * orows, orows), :],
                out_sems.at[grp],
            )

        for grp in range(N_OGRP):
            rows = pl.ds(grp * orows, orows)
            outv_ref[rows, :] = outv_ref[rows, :] + offset16
            out_copy(grp).start()
        for grp in range(N_OGRP):
            out_copy(grp).wait()

        for j in range(1, N_DEV):

            @pl.when(me < j)
            def _wait_send(j=j):
                rdma = pltpu.make_async_remote_copy(
                    src_ref=total_ref.at[0],
                    dst_ref=comm_ref.at[me],
                    send_sem=send_sems.at[j],
                    recv_sem=recv_sems.at[me],
                    device_id=(j,),
                    device_id_type=_DEV_ID_TYPE.MESH,
                )
                rdma.wait_send()

    return pl.pallas_call(
        body,
        out_shape=jax.ShapeDtypeStruct((m, n), jnp.bfloat16),
        in_specs=[pl.BlockSpec(memory_space=pl.ANY)],
        out_specs=pl.BlockSpec(memory_space=pl.ANY),
        scratch_shapes=[
            pltpu.VMEM((m, n), jnp.float32),
            pltpu.VMEM((m, n), jnp.bfloat16),
            pltpu.VMEM((1, n), jnp.float32),
            pltpu.VMEM((N_DEV, n), jnp.float32),
            pltpu.SemaphoreType.DMA((N_CHUNK,)),
            pltpu.SemaphoreType.DMA((N_OGRP,)),
            pltpu.SemaphoreType.DMA((N_DEV,)),
            pltpu.SemaphoreType.DMA((N_DEV,)),
        ],
        compiler_params=pltpu.CompilerParams(collective_id=0),
    )(x)


# device time: 13336 ns/iter; 1.2487x vs baseline; 1.0349x over previous
import jax
import jax.numpy as jnp
from jax import lax
from jax.experimental import pallas as pl
from jax.experimental.pallas import tpu as pltpu

N_DEV = 16
BLK = 128
N_CHUNK = 8
N_OGRP = 8

_DEV_ID_TYPE = getattr(pltpu, "DeviceIdType", None) or pl.DeviceIdType


def kernel(x):
    m, n = x.shape
    n_blk = m // BLK
    crows = m // N_CHUNK
    orows = m // N_OGRP

    def body(
        x_ref,
        out_ref,
        stage_ref,
        outv_ref,
        total_ref,
        comm_ref,
        in_sems,
        out_sems,
        send_sems,
        recv_sems,
    ):
        me = lax.axis_index("i")

        barrier_sem = pltpu.get_barrier_semaphore()
        for p in range(N_DEV):

            @pl.when(me != p)
            def _signal(p=p):
                pl.semaphore_signal(
                    barrier_sem,
                    inc=1,
                    device_id=(p,),
                    device_id_type=_DEV_ID_TYPE.MESH,
                )

        def in_copy(k):
            return pltpu.make_async_copy(
                x_ref.at[pl.ds(k * crows, crows), :],
                stage_ref.at[pl.ds(k * crows, crows), :],
                in_sems.at[k],
            )

        for k in range(N_CHUNK):
            in_copy(k).start()

        r = lax.broadcasted_iota(jnp.int32, (BLK, BLK), 0)
        c = lax.broadcasted_iota(jnp.int32, (BLK, BLK), 1)
        tri = (r >= c).astype(jnp.bfloat16)

        def cumsum_block(g, off):
            blk = stage_ref[pl.ds(g * BLK, BLK), :].astype(jnp.bfloat16)
            cs = jax.lax.dot(tri, blk, preferred_element_type=jnp.float32)
            outv_ref[pl.ds(g * BLK, BLK), :] = (cs + off).astype(jnp.bfloat16)
            return off + cs[BLK - 1 : BLK, :]

        off = jnp.zeros((1, n), jnp.float32)
        tot = jnp.zeros((1, n), jnp.float32)
        bpc = crows // BLK
        for k in range(N_CHUNK - 1):
            in_copy(k).wait()
            tot = tot + jnp.sum(
                stage_ref[pl.ds(k * crows, crows), :], axis=0, keepdims=True
            )
            for b in range(bpc):
                off = cumsum_block(k * bpc + b, off)

        last = N_CHUNK - 1
        in_copy(last).wait()
        tot = tot + jnp.sum(
            stage_ref[pl.ds(last * crows, crows), :], axis=0, keepdims=True
        )
        total_ref[0, :] = tot[0, :]

        pl.semaphore_wait(barrier_sem, N_DEV - 1)

        for j in range(1, N_DEV):

            @pl.when(me < j)
            def _send(j=j):
                rdma = pltpu.make_async_remote_copy(
                    src_ref=total_ref.at[0],
                    dst_ref=comm_ref.at[me],
                    send_sem=send_sems.at[j],
                    recv_sem=recv_sems.at[me],
                    device_id=(j,),
                    device_id_type=_DEV_ID_TYPE.MESH,
                )
                rdma.start()

        for b in range(bpc):
            off = cumsum_block(last * bpc + b, off)

        for k in range(N_DEV - 1):

            @pl.when(k < me)
            def _recv(k=k):
                rdma = pltpu.make_async_remote_copy(
                    src_ref=total_ref.at[0],
                    dst_ref=comm_ref.at[k],
                    send_sem=send_sems.at[k],
                    recv_sem=recv_sems.at[k],
                    device_id=(0,),
                    device_id_type=_DEV_ID_TYPE.MESH,
                )
                rdma.wait_recv()

        row_ids = lax.broadcasted_iota(jnp.int32, (N_DEV, n), 0)
        comm = comm_ref[:, :]
        offset16 = jnp.sum(
            jnp.where(row_ids < me, comm, jnp.zeros_like(comm)),
            axis=0,
            keepdims=True,
        ).astype(jnp.bfloat16)

        def out_copy(grp):
            return pltpu.make_async_copy(
                outv_ref.at[pl.ds(grp * orows, orows), :],
                out_ref.at[pl.ds(grp * orows, orows), :],
                out_sems.at[grp],
            )

        for grp in range(N_OGRP):
            rows = pl.ds(grp * orows, orows)
            outv_ref[rows, :] = outv_ref[rows, :] + offset16
            out_copy(grp).start()
        for grp in range(N_OGRP):
            out_copy(grp).wait()

        for j in range(1, N_DEV):

            @pl.when(me < j)
            def _wait_send(j=j):
                rdma = pltpu.make_async_remote_copy(
                    src_ref=total_ref.at[0],
                    dst_ref=comm_ref.at[me],
                    send_sem=send_sems.at[j],
                    recv_sem=recv_sems.at[me],
                    device_id=(j,),
                    device_id_type=_DEV_ID_TYPE.MESH,
                )
                rdma.wait_send()

    return pl.pallas_call(
        body,
        out_shape=jax.ShapeDtypeStruct((m, n), jnp.bfloat16),
        in_specs=[pl.BlockSpec(memory_space=pl.ANY)],
        out_specs=pl.BlockSpec(memory_space=pl.ANY),
        scratch_shapes=[
            pltpu.VMEM((m, n), jnp.float32),
            pltpu.VMEM((m, n), jnp.bfloat16),
            pltpu.VMEM((1, n), jnp.float32),
            pltpu.VMEM((N_DEV, n), jnp.float32),
            pltpu.SemaphoreType.DMA((N_CHUNK,)),
            pltpu.SemaphoreType.DMA((N_OGRP,)),
            pltpu.SemaphoreType.DMA((N_DEV,)),
            pltpu.SemaphoreType.DMA((N_DEV,)),
        ],
        compiler_params=pltpu.CompilerParams(collective_id=0),
    )(x)
